# baseline (speedup 1.0000x reference)
import jax
import jax.numpy as jnp
from jax import lax
from jax.experimental import pallas as pl
from jax.experimental.pallas import tpu as pltpu
from jax.experimental.pallas import tpu_sc as plsc

_NC, _NS = 2, 16
_NW = _NC * _NS
_C = 128
_NBUF = 5
_L16 = 16


def _make_body(seq_mod):
    def _gather_body(text_hbm, seq_hbm, table_hbm, out_hbm,
                     idx_v, seq_s, *bufs_and_sems):
        rows = bufs_and_sems[:_NBUF]
        gsem = bufs_and_sems[_NBUF:2 * _NBUF]
        ssem = bufs_and_sems[2 * _NBUF:3 * _NBUF]
        wid = lax.axis_index("s") * _NC + lax.axis_index("c")
        n = text_hbm.shape[0]
        b_per_w = n // _NW
        base = wid * b_per_w
        pltpu.sync_copy(seq_hbm, seq_s)
        pltpu.sync_copy(text_hbm.at[pl.ds(base, b_per_w)], idx_v)
        seq_len = seq_s[...]
        n_chunks = b_per_w // _C
        lane = lax.iota(jnp.int32, _L16)

        def _transform(j):
            off = j * _C
            for i in range(_C // _L16):
                o = off + i * _L16
                pos = base + o + lane
                col = lax.rem(pos, seq_mod)
                raw = idx_v[pl.ds(o, _L16)]
                idx_v[pl.ds(o, _L16)] = jnp.where(col < seq_len, raw + 1, 0)

        def _start_gather(b, j):
            pltpu.async_copy(
                table_hbm.at[idx_v.at[pl.ds(j * _C, _C)]], rows[b], gsem[b])

        def _wait_gather(b, j):
            pltpu.make_async_copy(
                table_hbm.at[idx_v.at[pl.ds(j * _C, _C)]], rows[b],
                gsem[b]).wait()

        def _start_scatter(b, j):
            pltpu.async_copy(
                rows[b], out_hbm.at[pl.ds(base + j * _C, _C)], ssem[b])

        def _wait_scatter(b, j):
            pltpu.make_async_copy(
                rows[b], out_hbm.at[pl.ds(base + j * _C, _C)], ssem[b]).wait()

        for b in range(_NBUF):
            _transform(b)
            _start_gather(b, b)

        @pl.loop(0, n_chunks - _NBUF, step=_NBUF)
        def _grp(j0):
            for b in range(_NBUF):
                j = j0 + b
                _transform(j + _NBUF)
                _wait_gather(b, j)
                _start_scatter(b, j)
                _wait_scatter(b, j)
                _start_gather(b, j + _NBUF)

        for b in range(_NBUF):
            j = n_chunks - _NBUF + b
            _wait_gather(b, j)
            _start_scatter(b, j)
        for b in range(_NBUF):
            j = n_chunks - _NBUF + b
            _wait_scatter(b, j)

    return _gather_body


def _embed_gather(table, text_flat, seq_arr, seq_mod):
    n = text_flat.shape[0]
    d = table.shape[1]
    b_per_w = n // _NW
    k = pl.kernel(
        _make_body(seq_mod),
        out_type=jax.ShapeDtypeStruct((n, d), table.dtype),
        mesh=plsc.VectorSubcoreMesh(
            core_axis_name="c", subcore_axis_name="s",
            num_cores=_NC, num_subcores=_NS),
        scratch_types=(
            [pltpu.VMEM((b_per_w,), jnp.int32),
             pltpu.VMEM((_L16,), jnp.int32)]
            + [pltpu.VMEM((_C, d), jnp.float32) for _ in range(_NBUF)]
            + [pltpu.SemaphoreType.DMA for _ in range(2 * _NBUF)]
        ),
    )
    return k(text_flat, seq_arr, table)


def kernel(text, seq_len, table):
    b, l = text.shape
    seq_arr = jnp.full((16,), seq_len, jnp.int32)
    out = _embed_gather(table, text.reshape(-1), seq_arr, l)
    return out.reshape(b, l, table.shape[1])
